# Initial kernel scaffold; baseline (speedup 1.0000x reference)
import jax
import jax.numpy as jnp
from jax.experimental import pallas as pl


def kernel(x):
    raise NotImplementedError("write your pallas kernel here")


# speedup vs baseline: 11.8092x; 11.8092x over previous
import jax
import jax.numpy as jnp
from jax import lax
from jax.experimental import pallas as pl

N_SAMPLES = 4


def _row_kernel(x_ref, o_ref):
    j = pl.program_id(0)
    f, i = 64, x_ref.shape[-1]
    s = i // f
    row = x_ref[0, 0, :]
    o_ref[0] = jnp.broadcast_to(row[None, :], (N_SAMPLES, i))
    chunk2 = x_ref[0, :, pl.ds(j * s, s)]
    m = jnp.max(chunk2)
    idx = lax.broadcasted_iota(jnp.int32, (1, s), 1)
    a = jnp.min(jnp.where(chunk2 == m, idx, s))
    cols = lax.broadcasted_iota(jnp.int32, (N_SAMPLES, s), 1)
    prow = lax.broadcasted_iota(jnp.int32, (N_SAMPLES, s), 0)
    t = a + prow
    t = jnp.where(t >= s, t - s, t)
    chunk_out = jnp.where(cols == t, m, jnp.float32(0.0))
    o_ref[0, :, pl.ds(j * s, s)] = chunk_out


def kernel(x):
    f, i = x.shape
    x3 = x.reshape(f, 1, i)
    return pl.pallas_call(
        _row_kernel,
        grid=(f,),
        in_specs=[pl.BlockSpec((1, 1, i), lambda j: (j, 0, 0))],
        out_specs=pl.BlockSpec((1, N_SAMPLES, i), lambda j: (j, 0, 0)),
        out_shape=jax.ShapeDtypeStruct((f, N_SAMPLES, i), x.dtype),
    )(x3)
